# baseline (speedup 1.0000x reference)
import functools

import jax
import jax.numpy as jnp
from jax import lax
from jax.experimental import pallas as pl
from jax.experimental.pallas import tpu as pltpu
from jax.experimental.pallas import tpu_sc as plsc

_B, _K, _N, _T, _M = 4, 4, 1024, 16, 5
_TM = _T * _M
_BT = _B * _T
_HS = 512
_LO = _N - _HS
_NW = 32
_RW = _HS // _NW
_CH = 256
_NCH = _LO // _CH


@functools.partial(
    pl.kernel,
    out_type=jax.ShapeDtypeStruct((_HS, _N), jnp.float32),
    mesh=plsc.VectorSubcoreMesh(core_axis_name="c", subcore_axis_name="s"),
    scratch_types=[pltpu.VMEM((_RW, _N), jnp.float32),
                   pltpu.VMEM((_RW, _N), jnp.float32),
                   pltpu.VMEM((_RW, _N), jnp.float32),
                   pltpu.VMEM((_RW, _N), jnp.float32),
                   pltpu.SemaphoreType.DMA],
)
def _sc_ksum(l_hbm, out_hbm, b0, b1, b2, b3, sem):
    wid = lax.axis_index("s") * 2 + lax.axis_index("c")
    base = _LO + wid * _RW
    bufs = (b0, b1, b2, b3)
    for k in range(_K):
        pltpu.make_async_copy(l_hbm.at[k, pl.ds(base, _RW), :], bufs[k],
                              sem).start()
    for k in range(_K):
        pltpu.make_async_copy(l_hbm.at[k, pl.ds(base, _RW), :], bufs[k],
                              sem).wait()

    def _chunk(c, _):
        sl = pl.ds(c * 16, 16)
        for r in range(_RW):
            b0[r, sl] = ((b0[r, sl] + b1[r, sl])
                         + (b2[r, sl] + b3[r, sl]))
        return _

    lax.fori_loop(0, _N // 16, _chunk, None)
    pltpu.sync_copy(b0, out_hbm.at[pl.ds(wid * _RW, _RW), :])


def _fold_weights(w1_ref, b1_ref, w2_ref, b2_ref):
    w1c = w1_ref[...].reshape(_TM, _M, _T).sum(axis=1)
    vt = jax.lax.dot_general(w1c, w2_ref[...],
                             (((0,), (1,)), ((), ())),
                             preferred_element_type=jnp.float32)
    beff = jax.lax.dot_general(b1_ref[...], w2_ref[...],
                               (((1,), (1,)), ((), ())),
                               preferred_element_type=jnp.float32)
    vb = jnp.tile(beff + b2_ref[...], (1, _B))
    return vt, vb


def _compute_z(x_ref, vt):
    return jnp.concatenate(
        [jnp.dot(x_ref[b, 0], vt, preferred_element_type=jnp.float32)
         for b in range(_B)], axis=1)


def _tc_low_kernel(l_hbm, x_ref, w1_ref, b1_ref, w2_ref, b2_ref,
                   out_ref, lbuf, z_ref, sem):
    for c in range(_NCH):
        pltpu.make_async_copy(
            l_hbm.at[:, pl.ds(c * _CH, _CH), :],
            lbuf.at[:, pl.ds(c * _CH, _CH), :],
            sem.at[c]).start()

    vt, vb = _fold_weights(w1_ref, b1_ref, w2_ref, b2_ref)
    z_ref[...] = _compute_z(x_ref, vt)

    for c in range(_NCH):
        pltpu.make_async_copy(
            l_hbm.at[:, pl.ds(c * _CH, _CH), :],
            lbuf.at[:, pl.ds(c * _CH, _CH), :],
            sem.at[c]).wait()
        rows = pl.ds(c * _CH, _CH)
        lsum = ((lbuf[0, rows, :] + lbuf[1, rows, :])
                + (lbuf[2, rows, :] + lbuf[3, rows, :]))
        acc = jnp.dot(lsum, z_ref[...],
                      preferred_element_type=jnp.float32) + vb
        for b in range(_B):
            out_ref[b, rows, :] = acc[:, b * _T:(b + 1) * _T]


def _tc_high_kernel(y_ref, lh_ref, x_ref, w1_ref, b1_ref, w2_ref, b2_ref,
                    out_ref, z_ref):
    out_ref[...] = y_ref[...]
    vt, vb = _fold_weights(w1_ref, b1_ref, w2_ref, b2_ref)
    z_ref[...] = _compute_z(x_ref, vt)
    acc = jnp.dot(lh_ref[...], z_ref[...],
                  preferred_element_type=jnp.float32) + vb
    for b in range(_B):
        out_ref[b, _LO:, :] = acc[:, b * _T:(b + 1) * _T]


def kernel(x, mul_L, W1, b1, W2, b2):
    b1r = b1.reshape(1, _TM)
    b2r = b2.reshape(1, _T)

    lsum_hi = _sc_ksum(mul_L)

    y_partial = pl.pallas_call(
        _tc_low_kernel,
        in_specs=[
            pl.BlockSpec(memory_space=pltpu.HBM),
            pl.BlockSpec((_B, 1, _N, _T), lambda: (0, 0, 0, 0)),
            pl.BlockSpec((_TM, _TM), lambda: (0, 0)),
            pl.BlockSpec((1, _TM), lambda: (0, 0)),
            pl.BlockSpec((_T, _TM), lambda: (0, 0)),
            pl.BlockSpec((1, _T), lambda: (0, 0)),
        ],
        out_specs=pl.BlockSpec((_B, _N, _T), lambda: (0, 0, 0)),
        out_shape=jax.ShapeDtypeStruct((_B, _N, _T), jnp.float32),
        scratch_shapes=[pltpu.VMEM((_K, _LO, _N), jnp.float32),
                        pltpu.VMEM((_N, _BT), jnp.float32),
                        pltpu.SemaphoreType.DMA((_NCH,))],
        compiler_params=pltpu.CompilerParams(
            vmem_limit_bytes=50 * 1024 * 1024),
    )(mul_L, x, W1, b1r, W2, b2r)

    return pl.pallas_call(
        _tc_high_kernel,
        in_specs=[
            pl.BlockSpec((_B, _N, _T), lambda: (0, 0, 0)),
            pl.BlockSpec((_HS, _N), lambda: (0, 0)),
            pl.BlockSpec((_B, 1, _N, _T), lambda: (0, 0, 0, 0)),
            pl.BlockSpec((_TM, _TM), lambda: (0, 0)),
            pl.BlockSpec((1, _TM), lambda: (0, 0)),
            pl.BlockSpec((_T, _TM), lambda: (0, 0)),
            pl.BlockSpec((1, _T), lambda: (0, 0)),
        ],
        out_specs=pl.BlockSpec((_B, _N, _T), lambda: (0, 0, 0)),
        out_shape=jax.ShapeDtypeStruct((_B, _N, _T), jnp.float32),
        scratch_shapes=[pltpu.VMEM((_N, _BT), jnp.float32)],
        input_output_aliases={0: 0},
        compiler_params=pltpu.CompilerParams(
            vmem_limit_bytes=50 * 1024 * 1024),
    )(y_partial, lsum_hi, x, W1, b1r, W2, b2r)


# speedup vs baseline: 2.1600x; 2.1600x over previous
import jax
import jax.numpy as jnp
from jax.experimental import pallas as pl
from jax.experimental.pallas import tpu as pltpu

_B, _K, _N, _T, _M = 4, 4, 1024, 16, 5
_TM = _T * _M
_BT = _B * _T
_BLK = 256


def _spectral_kernel(x_ref, w1_ref, b1_ref, w2_ref, b2_ref, l_ref,
                     out_ref, z_ref, vb_ref):
    i = pl.program_id(0)

    @pl.when(i == 0)
    def _init():
        w1c = w1_ref[...].reshape(_TM, _M, _T).sum(axis=1)
        vt = jax.lax.dot_general(w1c, w2_ref[...],
                                 (((0,), (1,)), ((), ())),
                                 preferred_element_type=jnp.float32)
        z_ref[...] = jnp.concatenate(
            [jnp.dot(x_ref[b, 0], vt, preferred_element_type=jnp.float32)
             for b in range(_B)], axis=1)
        beff = jax.lax.dot_general(b1_ref[...], w2_ref[...],
                                   (((1,), (1,)), ((), ())),
                                   preferred_element_type=jnp.float32)
        vb_ref[...] = jnp.tile(beff + b2_ref[...], (1, _B))

    z = z_ref[...]
    acc = ((jnp.dot(l_ref[0], z, preferred_element_type=jnp.float32)
            + jnp.dot(l_ref[1], z, preferred_element_type=jnp.float32))
           + (jnp.dot(l_ref[2], z, preferred_element_type=jnp.float32)
              + jnp.dot(l_ref[3], z, preferred_element_type=jnp.float32)))
    acc = acc + vb_ref[...]
    for b in range(_B):
        out_ref[b] = acc[:, b * _T:(b + 1) * _T]


def kernel(x, mul_L, W1, b1, W2, b2):
    return pl.pallas_call(
        _spectral_kernel,
        grid=(_N // _BLK,),
        in_specs=[
            pl.BlockSpec((_B, 1, _N, _T), lambda i: (0, 0, 0, 0)),
            pl.BlockSpec((_TM, _TM), lambda i: (0, 0)),
            pl.BlockSpec((1, _TM), lambda i: (0, 0)),
            pl.BlockSpec((_T, _TM), lambda i: (0, 0)),
            pl.BlockSpec((1, _T), lambda i: (0, 0)),
            pl.BlockSpec((_K, _BLK, _N), lambda i: (0, i, 0)),
        ],
        out_specs=pl.BlockSpec((_B, _BLK, _T), lambda i: (0, i, 0)),
        out_shape=jax.ShapeDtypeStruct((_B, _N, _T), jnp.float32),
        scratch_shapes=[pltpu.VMEM((_N, _BT), jnp.float32),
                        pltpu.VMEM((1, _BT), jnp.float32)],
    )(x, W1, b1.reshape(1, _TM), W2, b2.reshape(1, _T), mul_L)
